# Initial kernel scaffold; baseline (speedup 1.0000x reference)
import jax
import jax.numpy as jnp
from jax.experimental import pallas as pl


def kernel(x, modality_masks, W1, W3, W2, ln_w, ln_b):
    raise NotImplementedError("write your pallas kernel here")


# speedup vs baseline: 1.6100x; 1.6100x over previous
import functools

import jax
import jax.numpy as jnp
from jax import lax
from jax.experimental import pallas as pl
from jax.experimental.pallas import tpu as pltpu
from jax.experimental.pallas import tpu_sc as plsc

TOK_BLOCK = 512
HID_BLOCK = 512
ROWS_PER_CHUNK = 16


def _make_row_gather(n_out, dim, n_table, dtype):
    info = plsc.get_sparse_core_info()
    nc, ns = info.num_cores, info.num_subcores
    nw = nc * ns
    assert n_out % (8 * nw) == 0
    b_per_w = n_out // nw
    ch = ROWS_PER_CHUNK
    assert b_per_w % ch == 0
    nch = b_per_w // ch

    mesh = plsc.VectorSubcoreMesh(core_axis_name="c", subcore_axis_name="s")

    @functools.partial(
        pl.kernel,
        mesh=mesh,
        out_type=jax.ShapeDtypeStruct((n_out, dim), dtype),
        scratch_types=[
            pltpu.VMEM((b_per_w,), jnp.int32),
            pltpu.VMEM((2, ch, dim), dtype),
            pltpu.SemaphoreType.DMA,
            pltpu.SemaphoreType.DMA,
        ],
    )
    def gk(idx_hbm, table_hbm, out_hbm, idx_v, buf_v, sem0, sem1):
        wid = lax.axis_index("s") * nc + lax.axis_index("c")
        base = wid * b_per_w
        pltpu.sync_copy(idx_hbm.at[pl.ds(base, b_per_w)], idx_v)
        sems = (sem0, sem1)
        cp = pltpu.async_copy(
            table_hbm.at[idx_v.at[pl.ds(0, ch)]], buf_v.at[0], sems[0])
        for c in range(1, nch):
            nxt = pltpu.async_copy(
                table_hbm.at[idx_v.at[pl.ds(c * ch, ch)]],
                buf_v.at[c % 2], sems[c % 2])
            cp.wait()
            pltpu.sync_copy(buf_v.at[(c - 1) % 2],
                            out_hbm.at[pl.ds(base + (c - 1) * ch, ch)])
            cp = nxt
        cp.wait()
        pltpu.sync_copy(buf_v.at[(nch - 1) % 2],
                        out_hbm.at[pl.ds(base + (nch - 1) * ch, ch)])

    return gk


def _ffn_tc(xs, block_mod, W1, W3, W2, ln_w, ln_b):
    cap, dim = xs.shape
    nmod, hid, _ = W1.shape
    nb = cap // TOK_BLOCK
    nhb = hid // HID_BLOCK

    def body(bm_ref, x_ref, w1_ref, w3_ref, w2_ref, lnw_ref, lnb_ref,
             o_ref, acc_ref):
        hb = pl.program_id(1)
        x = x_ref[...]
        cdims = (((1,), (1,)), ((), ()))
        h1 = lax.dot_general(x, w1_ref[0], cdims,
                             preferred_element_type=jnp.float32)
        h3 = lax.dot_general(x, w3_ref[0], cdims,
                             preferred_element_type=jnp.float32)
        h = h1 * (h3 * jax.nn.sigmoid(h3))
        part = lax.dot_general(h, w2_ref[0], cdims,
                               preferred_element_type=jnp.float32)

        @pl.when(hb == 0)
        def _():
            acc_ref[...] = part

        @pl.when(hb != 0)
        def _():
            acc_ref[...] = acc_ref[...] + part

        @pl.when(hb == nhb - 1)
        def _():
            y = acc_ref[...]
            mean = jnp.mean(y, axis=1, keepdims=True)
            cen = y - mean
            var = jnp.mean(cen * cen, axis=1, keepdims=True)
            o_ref[...] = (cen * lax.rsqrt(var + 1e-5) * lnw_ref[0]
                          + lnb_ref[0])

    grid_spec = pltpu.PrefetchScalarGridSpec(
        num_scalar_prefetch=1,
        grid=(nb, nhb),
        in_specs=[
            pl.BlockSpec((TOK_BLOCK, dim), lambda tb, hb, bm: (tb, 0)),
            pl.BlockSpec((1, HID_BLOCK, dim),
                         lambda tb, hb, bm: (bm[tb], hb, 0)),
            pl.BlockSpec((1, HID_BLOCK, dim),
                         lambda tb, hb, bm: (bm[tb], hb, 0)),
            pl.BlockSpec((1, dim, HID_BLOCK),
                         lambda tb, hb, bm: (bm[tb], 0, hb)),
            pl.BlockSpec((1, 1, dim), lambda tb, hb, bm: (bm[tb], 0, 0)),
            pl.BlockSpec((1, 1, dim), lambda tb, hb, bm: (bm[tb], 0, 0)),
        ],
        out_specs=pl.BlockSpec((TOK_BLOCK, dim), lambda tb, hb, bm: (tb, 0)),
        scratch_shapes=[pltpu.VMEM((TOK_BLOCK, dim), jnp.float32)],
    )
    return pl.pallas_call(
        body,
        grid_spec=grid_spec,
        out_shape=jax.ShapeDtypeStruct((cap, dim), jnp.float32),
        compiler_params=pltpu.CompilerParams(
            dimension_semantics=("arbitrary", "arbitrary")),
    )(block_mod, xs, W1, W3, W2, ln_w[:, None, :], ln_b[:, None, :])


def kernel(x, modality_masks, W1, W3, W2, ln_w, ln_b):
    ntok, dim = x.shape
    blk = TOK_BLOCK
    cap = ntok + blk

    m0 = modality_masks[0]
    m0i = m0.astype(jnp.int32)
    c0 = jnp.sum(m0i)
    align = ((c0 + blk - 1) // blk) * blk
    r0 = jnp.cumsum(m0i) - 1
    r1 = jnp.cumsum(1 - m0i) - 1
    dest = jnp.where(m0, r0, align + r1).astype(jnp.int32)
    src = jnp.zeros((cap,), jnp.int32).at[dest].set(
        jnp.arange(ntok, dtype=jnp.int32))
    nb = cap // blk
    block_mod = (jnp.arange(nb, dtype=jnp.int32) * blk >= align).astype(
        jnp.int32)

    xs = _make_row_gather(cap, dim, ntok, x.dtype)(src, x)
    y = _ffn_tc(xs, block_mod, W1, W3, W2, ln_w, ln_b)
    out = _make_row_gather(ntok, dim, cap, x.dtype)(dest, y)
    return out
